# baseline (speedup 1.0000x reference)
import functools

import jax
import jax.numpy as jnp
from jax import lax
from jax.experimental import pallas as pl
from jax.experimental.pallas import tpu as pltpu
from jax.experimental.pallas import tpu_sc as plsc

E = 8
K = 2
TM = 128
NC = 2
NS = 16
NW = NC * NS


def _exclusive_prefix(oh):
    T = oh.shape[0]
    p = oh
    sh = 1
    while sh < T:
        shifted = jnp.concatenate(
            [jnp.zeros((sh, oh.shape[1]), oh.dtype), p[:-sh]], axis=0)
        p = p + shifted
        sh *= 2
    return p - oh


def _routing_kernel(nt, x_ref, wg_ref, d0_ref, d1_ref, w0_ref,
                    w1_ref, meta_ref):
    x = x_ref[...]
    logits = lax.dot_general(x, wg_ref[...], (((1,), (1,)), ((), ())),
                             preferred_element_type=jnp.float32)
    iota = lax.broadcasted_iota(jnp.int32, logits.shape, 1)
    m1 = jnp.max(logits, axis=1, keepdims=True)
    i1 = jnp.min(jnp.where(logits == m1, iota, E), axis=1, keepdims=True)
    masked = jnp.where(iota == i1, jnp.float32(-1e30), logits)
    m2 = jnp.max(masked, axis=1, keepdims=True)
    i2 = jnp.min(jnp.where((logits == m2) & (iota != i1), iota, E),
                 axis=1, keepdims=True)
    e2w = jnp.exp(m2 - m1)
    denom = 1.0 + e2w
    nw1 = 1.0 / denom
    nw2 = e2w / denom

    oh1 = (iota == i1).astype(jnp.float32)
    oh2 = (iota == i2).astype(jnp.float32)
    c1 = jnp.sum(oh1, axis=0, keepdims=True)
    c2 = jnp.sum(oh2, axis=0, keepdims=True)
    cnt = c1 + c2
    p1 = _exclusive_prefix(oh1)
    p2 = _exclusive_prefix(oh2)
    rank1 = jnp.sum(p1 * oh1, axis=1, keepdims=True)
    rank2 = jnp.sum((p2 + c1) * oh2, axis=1, keepdims=True)

    ntiles = jnp.floor((cnt + (TM - 1)) / TM)
    padded = ntiles * TM
    lane_r = lax.broadcasted_iota(jnp.int32, (E, E), 0)
    lane_c = lax.broadcasted_iota(jnp.int32, (E, E), 1)
    su = (lane_r < lane_c).astype(jnp.float32)
    start = lax.dot_general(padded, su, (((1,), (0,)), ((), ())),
                            preferred_element_type=jnp.float32)
    ts = lax.dot_general(ntiles, su, (((1,), (0,)), ((), ())),
                         preferred_element_type=jnp.float32)
    total_tiles = jnp.sum(ntiles)

    base1 = jnp.sum(oh1 * start, axis=1, keepdims=True)
    base2 = jnp.sum(oh2 * start, axis=1, keepdims=True)
    d0_ref[...] = (base1 + rank1).astype(jnp.int32)
    d1_ref[...] = (base2 + rank2).astype(jnp.int32)
    w0_ref[...] = jnp.broadcast_to(nw1, (nw1.shape[0], 128))
    w1_ref[...] = jnp.broadcast_to(nw2, (nw2.shape[0], 128))

    jt = lax.broadcasted_iota(jnp.int32, (nt, 1), 0).astype(jnp.float32)
    gid = jnp.sum((ts <= jt).astype(jnp.int32), axis=1, keepdims=True) - 1
    act = (jt < total_tiles).astype(jnp.int32)
    clamp = jnp.minimum(jt, total_tiles - 1.0).astype(jnp.int32)
    meta_ref[...] = jnp.concatenate([gid, act, clamp], axis=1)


def _routing(xf, Wg, nt, interpret=False):
    T, H = xf.shape
    return pl.pallas_call(
        functools.partial(_routing_kernel, nt),
        in_specs=[
            pl.BlockSpec((T, H), lambda: (0, 0)),
            pl.BlockSpec((E, H), lambda: (0, 0)),
        ],
        out_specs=[
            pl.BlockSpec((T, 1), lambda: (0, 0)),
            pl.BlockSpec((T, 1), lambda: (0, 0)),
            pl.BlockSpec((T, 128), lambda: (0, 0)),
            pl.BlockSpec((T, 128), lambda: (0, 0)),
            pl.BlockSpec((nt, 3), lambda: (0, 0)),
        ],
        out_shape=[
            jax.ShapeDtypeStruct((T, 1), jnp.int32),
            jax.ShapeDtypeStruct((T, 1), jnp.int32),
            jax.ShapeDtypeStruct((T, 128), jnp.float32),
            jax.ShapeDtypeStruct((T, 128), jnp.float32),
            jax.ShapeDtypeStruct((nt, 3), jnp.int32),
        ],
        interpret=interpret,
    )(xf, Wg)


def _dispatch(xf, d0, d1, w0e, w1e, ntot):
    T, H = xf.shape
    tpw = T // NW
    mesh = plsc.VectorSubcoreMesh(core_axis_name="c", subcore_axis_name="s")

    @functools.partial(
        pl.kernel,
        out_type=[
            jax.ShapeDtypeStruct((ntot, H), jnp.float32),
            jax.ShapeDtypeStruct((ntot, 128), jnp.float32),
        ],
        mesh=mesh,
        scratch_types=[
            pltpu.VMEM((tpw, H), jnp.float32),
            pltpu.VMEM((tpw,), jnp.int32),
            pltpu.VMEM((tpw,), jnp.int32),
            pltpu.VMEM((tpw, 128), jnp.float32),
            pltpu.VMEM((tpw, 128), jnp.float32),
            pltpu.SemaphoreType.DMA,
            pltpu.SemaphoreType.DMA,
        ],
    )
    def k(x_hbm, d0_hbm, d1_hbm, w0_hbm, w1_hbm, xs_hbm, ws_hbm,
          xbuf, i0, i1, wb0, wb1, sem, semw):
        wid = lax.axis_index("s") * NC + lax.axis_index("c")
        base = wid * tpw
        r0 = pltpu.async_copy(d0_hbm.at[pl.ds(base, tpw)], i0, semw)
        r1 = pltpu.async_copy(d1_hbm.at[pl.ds(base, tpw)], i1, semw)
        r2 = pltpu.async_copy(x_hbm.at[pl.ds(base, tpw)], xbuf, sem)
        r3 = pltpu.async_copy(w0_hbm.at[pl.ds(base, tpw)], wb0, semw)
        r4 = pltpu.async_copy(w1_hbm.at[pl.ds(base, tpw)], wb1, semw)
        r0.wait(); r1.wait(); r3.wait(); r4.wait(); r2.wait()
        c0 = pltpu.async_copy(xbuf, xs_hbm.at[i0], sem)
        c1 = pltpu.async_copy(xbuf, xs_hbm.at[i1], sem)
        c2 = pltpu.async_copy(wb0, ws_hbm.at[i0], semw)
        c3 = pltpu.async_copy(wb1, ws_hbm.at[i1], semw)
        c0.wait(); c1.wait(); c2.wait(); c3.wait()

    return k(xf, d0, d1, w0e, w1e)


def _ffn_kernel(meta_ref, xs_ref, ws_ref, w1_ref, w3_ref, w2_ref,
                out_ref):
    i = pl.program_id(0)

    @pl.when(meta_ref[i, 1] == 1)
    def _():
        x = xs_ref[...]
        g = lax.dot_general(x, w1_ref[0], (((1,), (1,)), ((), ())),
                            preferred_element_type=jnp.float32)
        u = lax.dot_general(x, w3_ref[0], (((1,), (1,)), ((), ())),
                            preferred_element_type=jnp.float32)
        h = (g * lax.logistic(g)) * u * ws_ref[:, 0:1]
        out_ref[...] = lax.dot_general(h, w2_ref[0], (((1,), (1,)), ((), ())),
                                       preferred_element_type=jnp.float32)


def _grouped_ffn(xs, ws, W1, W3, W2, meta, nt, interpret=False):
    ntot, H = xs.shape
    DFF = W1.shape[1]
    grid_spec = pltpu.PrefetchScalarGridSpec(
        num_scalar_prefetch=1,
        grid=(nt,),
        in_specs=[
            pl.BlockSpec((TM, H), lambda i, m: (m[i, 2], 0)),
            pl.BlockSpec((TM, 128), lambda i, m: (m[i, 2], 0)),
            pl.BlockSpec((1, DFF, H), lambda i, m: (m[i, 0], 0, 0)),
            pl.BlockSpec((1, DFF, H), lambda i, m: (m[i, 0], 0, 0)),
            pl.BlockSpec((1, H, DFF), lambda i, m: (m[i, 0], 0, 0)),
        ],
        out_specs=pl.BlockSpec((TM, H), lambda i, m: (m[i, 2], 0)),
    )
    return pl.pallas_call(
        _ffn_kernel,
        grid_spec=grid_spec,
        out_shape=jax.ShapeDtypeStruct((ntot, H), jnp.float32),
        compiler_params=pltpu.CompilerParams(
            dimension_semantics=("arbitrary",)),
        interpret=interpret,
    )(meta, xs, ws, W1, W3, W2)


def _combine(os_, d0, d1, T):
    ntot, H = os_.shape
    tpw = T // NW
    ck = 16
    nchunks = tpw // ck
    mesh = plsc.VectorSubcoreMesh(core_axis_name="c", subcore_axis_name="s")

    @functools.partial(
        pl.kernel,
        out_type=jax.ShapeDtypeStruct((T, H), jnp.float32),
        mesh=mesh,
        scratch_types=[
            pltpu.VMEM((ck, H), jnp.float32),
            pltpu.VMEM((ck, H), jnp.float32),
            pltpu.VMEM((ck, H), jnp.float32),
            pltpu.VMEM((ck, H), jnp.float32),
            pltpu.VMEM((ck, H), jnp.float32),
            pltpu.VMEM((ck,), jnp.int32),
            pltpu.VMEM((ck,), jnp.int32),
            pltpu.VMEM((ck,), jnp.int32),
            pltpu.VMEM((ck,), jnp.int32),
            pltpu.SemaphoreType.DMA,
            pltpu.SemaphoreType.DMA,
        ],
    )
    def k(os_hbm, d0_hbm, d1_hbm, y_hbm,
          b00, b10, b01, b11, yb, i00, i10, i01, i11, s0, s1):
        wid = lax.axis_index("s") * NC + lax.axis_index("c")
        base = wid * tpw
        bufs = ((b00, b10, i00, i10, s0), (b01, b11, i01, i11, s1))

        def fire(c):
            b0, b1, i0, i1, s = bufs[c % 2]
            pltpu.sync_copy(d0_hbm.at[pl.ds(base + c * ck, ck)], i0)
            pltpu.sync_copy(d1_hbm.at[pl.ds(base + c * ck, ck)], i1)
            h0 = pltpu.async_copy(os_hbm.at[i0], b0, s)
            h1 = pltpu.async_copy(os_hbm.at[i1], b1, s)
            return h0, h1

        handles = [None, None]
        handles[0] = fire(0)
        for c in range(nchunks):
            if c + 1 < nchunks:
                handles[(c + 1) % 2] = fire(c + 1)
            h0, h1 = handles[c % 2]
            h0.wait()
            h1.wait()
            b0, b1, _, _, _ = bufs[c % 2]

            def row(t, _):
                for sidx in range(H // 16):
                    sl = pl.ds(sidx * 16, 16)
                    yb[t, sl] = b0[t, sl] + b1[t, sl]
                return 0

            lax.fori_loop(0, ck, row, 0)
            pltpu.sync_copy(yb, y_hbm.at[pl.ds(base + c * ck, ck)])

    return k(os_, d0, d1)


def kernel(x, Wg, W1, W2, W3, Ws1, Ws2, Ws3):
    B, S, H = x.shape
    T = B * S
    nt = (K * T) // TM + (E - 1)
    ntot = nt * TM
    xf = x.reshape(T, H)
    d0, d1, w0e, w1e, meta = _routing(xf, Wg, nt)
    d0 = d0.reshape(T)
    d1 = d1.reshape(T)
    xs, ws = _dispatch(xf, d0, d1, w0e, w1e, ntot)
    os_ = _grouped_ffn(xs, ws, W1, W3, W2, meta, nt)
    y = _combine(os_, d0, d1, T)
    return y.reshape(B, S, H)


# speedup vs baseline: 1.1877x; 1.1877x over previous
import functools

import jax
import jax.numpy as jnp
from jax import lax
from jax.experimental import pallas as pl
from jax.experimental.pallas import tpu as pltpu
from jax.experimental.pallas import tpu_sc as plsc

E = 8
K = 2
TM = 256
NC = 2
NS = 16
NW = NC * NS


def _exclusive_prefix(oh):
    T = oh.shape[0]
    p = oh
    sh = 1
    while sh < T:
        shifted = jnp.concatenate(
            [jnp.zeros((sh, oh.shape[1]), oh.dtype), p[:-sh]], axis=0)
        p = p + shifted
        sh *= 2
    return p - oh


def _routing_kernel(nt, x_ref, wg_ref, d0_ref, d1_ref, w0_ref,
                    w1_ref, meta_ref):
    x = x_ref[...]
    logits = lax.dot_general(x, wg_ref[...], (((1,), (1,)), ((), ())),
                             preferred_element_type=jnp.float32)
    iota = lax.broadcasted_iota(jnp.int32, logits.shape, 1)
    m1 = jnp.max(logits, axis=1, keepdims=True)
    i1 = jnp.min(jnp.where(logits == m1, iota, E), axis=1, keepdims=True)
    masked = jnp.where(iota == i1, jnp.float32(-1e30), logits)
    m2 = jnp.max(masked, axis=1, keepdims=True)
    i2 = jnp.min(jnp.where((logits == m2) & (iota != i1), iota, E),
                 axis=1, keepdims=True)
    e2w = jnp.exp(m2 - m1)
    denom = 1.0 + e2w
    nw1 = 1.0 / denom
    nw2 = e2w / denom

    oh1 = (iota == i1).astype(jnp.float32)
    oh2 = (iota == i2).astype(jnp.float32)
    c1 = jnp.sum(oh1, axis=0, keepdims=True)
    c2 = jnp.sum(oh2, axis=0, keepdims=True)
    cnt = c1 + c2
    p1 = _exclusive_prefix(oh1)
    p2 = _exclusive_prefix(oh2)
    rank1 = jnp.sum(p1 * oh1, axis=1, keepdims=True)
    rank2 = jnp.sum((p2 + c1) * oh2, axis=1, keepdims=True)

    ntiles = jnp.floor((cnt + (TM - 1)) / TM)
    padded = ntiles * TM
    lane_r = lax.broadcasted_iota(jnp.int32, (E, E), 0)
    lane_c = lax.broadcasted_iota(jnp.int32, (E, E), 1)
    su = (lane_r < lane_c).astype(jnp.float32)
    start = lax.dot_general(padded, su, (((1,), (0,)), ((), ())),
                            preferred_element_type=jnp.float32)
    ts = lax.dot_general(ntiles, su, (((1,), (0,)), ((), ())),
                         preferred_element_type=jnp.float32)
    total_tiles = jnp.sum(ntiles)

    base1 = jnp.sum(oh1 * start, axis=1, keepdims=True)
    base2 = jnp.sum(oh2 * start, axis=1, keepdims=True)
    d0_ref[...] = (base1 + rank1).astype(jnp.int32)
    d1_ref[...] = (base2 + rank2).astype(jnp.int32)
    w0_ref[...] = jnp.broadcast_to(nw1, (nw1.shape[0], 128))
    w1_ref[...] = jnp.broadcast_to(nw2, (nw2.shape[0], 128))

    jt = lax.broadcasted_iota(jnp.int32, (nt, 1), 0).astype(jnp.float32)
    gid = jnp.sum((ts <= jt).astype(jnp.int32), axis=1, keepdims=True) - 1
    act = (jt < total_tiles).astype(jnp.int32)
    clamp = jnp.minimum(jt, total_tiles - 1.0).astype(jnp.int32)
    meta_ref[...] = jnp.concatenate([gid, act, clamp], axis=1)


def _routing(xf, Wg, nt, interpret=False):
    T, H = xf.shape
    return pl.pallas_call(
        functools.partial(_routing_kernel, nt),
        in_specs=[
            pl.BlockSpec((T, H), lambda: (0, 0)),
            pl.BlockSpec((E, H), lambda: (0, 0)),
        ],
        out_specs=[
            pl.BlockSpec((T, 1), lambda: (0, 0)),
            pl.BlockSpec((T, 1), lambda: (0, 0)),
            pl.BlockSpec((T, 128), lambda: (0, 0)),
            pl.BlockSpec((T, 128), lambda: (0, 0)),
            pl.BlockSpec((nt, 3), lambda: (0, 0)),
        ],
        out_shape=[
            jax.ShapeDtypeStruct((T, 1), jnp.int32),
            jax.ShapeDtypeStruct((T, 1), jnp.int32),
            jax.ShapeDtypeStruct((T, 128), jnp.float32),
            jax.ShapeDtypeStruct((T, 128), jnp.float32),
            jax.ShapeDtypeStruct((nt, 3), jnp.int32),
        ],
        interpret=interpret,
    )(xf, Wg)


def _dispatch(xf, d0, d1, w0e, w1e, ntot):
    T, H = xf.shape
    tpw = T // NW
    mesh = plsc.VectorSubcoreMesh(core_axis_name="c", subcore_axis_name="s")

    @functools.partial(
        pl.kernel,
        out_type=[
            jax.ShapeDtypeStruct((ntot, H), jnp.float32),
            jax.ShapeDtypeStruct((ntot, 128), jnp.float32),
        ],
        mesh=mesh,
        scratch_types=[
            pltpu.VMEM((tpw, H), jnp.float32),
            pltpu.VMEM((tpw,), jnp.int32),
            pltpu.VMEM((tpw,), jnp.int32),
            pltpu.VMEM((tpw, 128), jnp.float32),
            pltpu.VMEM((tpw, 128), jnp.float32),
            pltpu.SemaphoreType.DMA,
            pltpu.SemaphoreType.DMA,
        ],
    )
    def k(x_hbm, d0_hbm, d1_hbm, w0_hbm, w1_hbm, xs_hbm, ws_hbm,
          xbuf, i0, i1, wb0, wb1, sem, semw):
        wid = lax.axis_index("s") * NC + lax.axis_index("c")
        base = wid * tpw
        r0 = pltpu.async_copy(d0_hbm.at[pl.ds(base, tpw)], i0, semw)
        r1 = pltpu.async_copy(d1_hbm.at[pl.ds(base, tpw)], i1, semw)
        r2 = pltpu.async_copy(x_hbm.at[pl.ds(base, tpw)], xbuf, sem)
        r3 = pltpu.async_copy(w0_hbm.at[pl.ds(base, tpw)], wb0, semw)
        r4 = pltpu.async_copy(w1_hbm.at[pl.ds(base, tpw)], wb1, semw)
        r0.wait(); r1.wait(); r3.wait(); r4.wait(); r2.wait()
        c0 = pltpu.async_copy(xbuf, xs_hbm.at[i0], sem)
        c1 = pltpu.async_copy(xbuf, xs_hbm.at[i1], sem)
        c2 = pltpu.async_copy(wb0, ws_hbm.at[i0], semw)
        c3 = pltpu.async_copy(wb1, ws_hbm.at[i1], semw)
        c0.wait(); c1.wait(); c2.wait(); c3.wait()

    return k(xf, d0, d1, w0e, w1e)


def _ffn_kernel(meta_ref, xs_ref, ws_ref, w1_ref, w3_ref, w2_ref,
                out_ref):
    i = pl.program_id(0)

    @pl.when(meta_ref[i, 1] == 1)
    def _():
        x = xs_ref[...]
        g = lax.dot_general(x, w1_ref[0], (((1,), (1,)), ((), ())),
                            preferred_element_type=jnp.float32)
        u = lax.dot_general(x, w3_ref[0], (((1,), (1,)), ((), ())),
                            preferred_element_type=jnp.float32)
        h = (g * lax.logistic(g)) * u * ws_ref[:, 0:1]
        out_ref[...] = lax.dot_general(h, w2_ref[0], (((1,), (1,)), ((), ())),
                                       preferred_element_type=jnp.float32)


def _grouped_ffn(xs, ws, W1, W3, W2, meta, nt, interpret=False):
    ntot, H = xs.shape
    DFF = W1.shape[1]
    grid_spec = pltpu.PrefetchScalarGridSpec(
        num_scalar_prefetch=1,
        grid=(nt,),
        in_specs=[
            pl.BlockSpec((TM, H), lambda i, m: (m[i, 2], 0)),
            pl.BlockSpec((TM, 128), lambda i, m: (m[i, 2], 0)),
            pl.BlockSpec((1, DFF, H), lambda i, m: (m[i, 0], 0, 0)),
            pl.BlockSpec((1, DFF, H), lambda i, m: (m[i, 0], 0, 0)),
            pl.BlockSpec((1, H, DFF), lambda i, m: (m[i, 0], 0, 0)),
        ],
        out_specs=pl.BlockSpec((TM, H), lambda i, m: (m[i, 2], 0)),
    )
    return pl.pallas_call(
        _ffn_kernel,
        grid_spec=grid_spec,
        out_shape=jax.ShapeDtypeStruct((ntot, H), jnp.float32),
        compiler_params=pltpu.CompilerParams(
            dimension_semantics=("arbitrary",)),
        interpret=interpret,
    )(meta, xs, ws, W1, W3, W2)


def _combine(os_, d0, d1, T):
    ntot, H = os_.shape
    tpw = T // NW
    ck = 16
    nchunks = tpw // ck
    mesh = plsc.VectorSubcoreMesh(core_axis_name="c", subcore_axis_name="s")

    @functools.partial(
        pl.kernel,
        out_type=jax.ShapeDtypeStruct((T, H), jnp.float32),
        mesh=mesh,
        scratch_types=[
            pltpu.VMEM((ck, H), jnp.float32),
            pltpu.VMEM((ck, H), jnp.float32),
            pltpu.VMEM((ck, H), jnp.float32),
            pltpu.VMEM((ck, H), jnp.float32),
            pltpu.VMEM((ck, H), jnp.float32),
            pltpu.VMEM((ck,), jnp.int32),
            pltpu.VMEM((ck,), jnp.int32),
            pltpu.VMEM((ck,), jnp.int32),
            pltpu.VMEM((ck,), jnp.int32),
            pltpu.SemaphoreType.DMA,
            pltpu.SemaphoreType.DMA,
        ],
    )
    def k(os_hbm, d0_hbm, d1_hbm, y_hbm,
          b00, b10, b01, b11, yb, i00, i10, i01, i11, s0, s1):
        wid = lax.axis_index("s") * NC + lax.axis_index("c")
        base = wid * tpw
        bufs = ((b00, b10, i00, i10, s0), (b01, b11, i01, i11, s1))

        def fire(c):
            b0, b1, i0, i1, s = bufs[c % 2]
            pltpu.sync_copy(d0_hbm.at[pl.ds(base + c * ck, ck)], i0)
            pltpu.sync_copy(d1_hbm.at[pl.ds(base + c * ck, ck)], i1)
            h0 = pltpu.async_copy(os_hbm.at[i0], b0, s)
            h1 = pltpu.async_copy(os_hbm.at[i1], b1, s)
            return h0, h1

        handles = [None, None]
        handles[0] = fire(0)
        for c in range(nchunks):
            if c + 1 < nchunks:
                handles[(c + 1) % 2] = fire(c + 1)
            h0, h1 = handles[c % 2]
            h0.wait()
            h1.wait()
            b0, b1, _, _, _ = bufs[c % 2]

            def row(t, _):
                for sidx in range(H // 16):
                    sl = pl.ds(sidx * 16, 16)
                    yb[t, sl] = b0[t, sl] + b1[t, sl]
                return 0

            lax.fori_loop(0, ck, row, 0)
            pltpu.sync_copy(yb, y_hbm.at[pl.ds(base + c * ck, ck)])

    return k(os_, d0, d1)


def kernel(x, Wg, W1, W2, W3, Ws1, Ws2, Ws3):
    B, S, H = x.shape
    T = B * S
    nt = (K * T) // TM + (E - 1)
    ntot = nt * TM
    xf = x.reshape(T, H)
    d0, d1, w0e, w1e, meta = _routing(xf, Wg, nt)
    d0 = d0.reshape(T)
    d1 = d1.reshape(T)
    xs, ws = _dispatch(xf, d0, d1, w0e, w1e, ntot)
    os_ = _grouped_ffn(xs, ws, W1, W3, W2, meta, nt)
    y = _combine(os_, d0, d1, T)
    return y.reshape(B, S, H)
